# Initial kernel scaffold; baseline (speedup 1.0000x reference)
import jax
import jax.numpy as jnp
from jax.experimental import pallas as pl


def kernel(x, table, W, b):
    raise NotImplementedError("write your pallas kernel here")


# speedup vs baseline: 15.3142x; 15.3142x over previous
import functools
import math

import jax
import jax.numpy as jnp
from jax import lax
from jax.experimental import pallas as pl
from jax.experimental.pallas import tpu as pltpu
from jax.experimental.pallas import tpu_sc as plsc

_NC = 2
_NS = 16
_NW = _NC * _NS

_ROWB = 128
_G = 8


def _gather_body(table_hbm, idx_hbm, out_hbm, idx_v, rows_v, sem):
    wid = lax.axis_index("s") * _NC + lax.axis_index("c")
    nrows = idx_hbm.shape[0]
    rows_per_w = nrows // _NW
    steps = rows_per_w // _G

    def step(i, carry):
        base = wid * rows_per_w + i * _G
        pltpu.sync_copy(idx_hbm.at[pl.ds(base, _G)], idx_v)
        copies = [
            pltpu.async_copy(table_hbm.at[idx_v.at[j]], rows_v.at[j], sem)
            for j in range(_G)
        ]
        for c in copies:
            c.wait()
        pltpu.sync_copy(rows_v, out_hbm.at[pl.ds(base, _G)])
        return carry

    lax.fori_loop(0, steps, step, 0)


def _make_gather(nrows, embed):
    mesh = plsc.VectorSubcoreMesh(core_axis_name="c", subcore_axis_name="s")
    return pl.kernel(
        _gather_body,
        out_type=jax.ShapeDtypeStruct((nrows, _ROWB, embed), jnp.float32),
        mesh=mesh,
        scratch_types=[
            pltpu.VMEM((_G, _ROWB), jnp.int32),
            pltpu.VMEM((_G, _ROWB, embed), jnp.float32),
            pltpu.SemaphoreType.DMA,
        ],
        compiler_params=pltpu.CompilerParams(use_tc_tiling_on_sc=False),
    )


def _proj_body(emb_ref, w_ref, b_ref, out_ref):
    acc = jnp.dot(emb_ref[...], w_ref[...], preferred_element_type=jnp.float32)
    out_ref[...] = (acc + b_ref[...]) * math.sqrt(float(w_ref.shape[1]))


def _proj(emb, w, b2):
    n, e = emb.shape
    d = w.shape[1]
    blk = 4096
    return pl.pallas_call(
        _proj_body,
        grid=(n // blk,),
        in_specs=[
            pl.BlockSpec((blk, e), lambda i: (i, 0)),
            pl.BlockSpec((e, d), lambda i: (0, 0)),
            pl.BlockSpec((1, d), lambda i: (0, 0)),
        ],
        out_specs=pl.BlockSpec((blk, d), lambda i: (i, 0)),
        out_shape=jax.ShapeDtypeStruct((n, d), jnp.float32),
        compiler_params=pltpu.CompilerParams(
            dimension_semantics=("arbitrary",),
        ),
    )(emb, w, b2)


def kernel(x, table, W, b):
    batch, hist = x.shape
    n = batch * hist
    embed = table.shape[1]
    d_model = W.shape[1]

    idx = x.reshape(n // _ROWB, _ROWB).astype(jnp.int32)
    emb = _make_gather(n // _ROWB, embed)(table, idx)
    out = _proj(emb.reshape(n, embed), W, b.reshape(1, d_model))
    return out.reshape(batch, hist, d_model)
